# Initial kernel scaffold; baseline (speedup 1.0000x reference)
import jax
import jax.numpy as jnp
from jax.experimental import pallas as pl


def kernel(x):
    raise NotImplementedError("write your pallas kernel here")


# speedup vs baseline: 32.7822x; 32.7822x over previous
import functools

import jax
import jax.numpy as jnp
from jax import lax
from jax.experimental import pallas as pl
from jax.experimental.pallas import tpu as pltpu
from jax.experimental.pallas import tpu_sc as plsc

_B = 4
_N = 4096
_H = 0.0005
_K = 8
_NC = 2
_NS = 16
_NW = _NC * _NS
_ROWS_PER_W = (_B * _N) // _NW
_GROUPS = _ROWS_PER_W // 16
_CHUNKS = _N // 16
_TILES_PER_B = _NW // _B
_BIG = 3.0e38


def _body(xr_hbm, sq_hbm, out_hbm, x0v, x1v, x2v, sqv, ans_v, corr_s):
    cid = lax.axis_index("c")
    sid = lax.axis_index("s")
    wid = sid * _NC + cid
    b = wid // _TILES_PER_B
    seg = wid % _TILES_PER_B

    base = b * (3 * _N)
    pltpu.sync_copy(xr_hbm.at[pl.ds(base, _N)], x0v)
    pltpu.sync_copy(xr_hbm.at[pl.ds(base + _N, _N)], x1v)
    pltpu.sync_copy(xr_hbm.at[pl.ds(base + 2 * _N, _N)], x2v)
    pltpu.sync_copy(sq_hbm.at[pl.ds(b * _N, _N)], sqv)

    corr_s[0] = jnp.float32(0.0)
    lane = jax.lax.iota(jnp.int32, 16)

    def dist_chunk(q0, q1, q2, sqq, jo):
        r0 = x0v[pl.ds(jo, 16)]
        r1 = x1v[pl.ds(jo, 16)]
        r2 = x2v[pl.ds(jo, 16)]
        sqj = sqv[pl.ds(jo, 16)]
        dot = (q0 * r0 + q1 * r1) + q2 * r2
        dd = (sqq + sqj) - 2.0 * dot
        return jnp.maximum(dd, 0.0)

    def row_step(qv0, qv1, qv2, sqg, r, carry):
        acc, msum = carry
        ridx = jnp.full((16,), 0, jnp.int32) + r
        q0 = jnp.take_along_axis(qv0, ridx, axis=0)
        q1 = jnp.take_along_axis(qv1, ridx, axis=0)
        q2 = jnp.take_along_axis(qv2, ridx, axis=0)
        sqq = jnp.take_along_axis(sqg, ridx, axis=0)

        def chunk_step(jb, carry_c):
            s, c, dmin = carry_c
            dd = dist_chunk(q0, q1, q2, sqq, jb * 16)
            m = dd < _H
            s = s + jnp.where(m, _H - dd, 0.0)
            c = c + jnp.where(m, 1.0, 0.0)
            dmin = jnp.minimum(dmin, dd)
            return s, c, dmin

        zeros = jnp.zeros((16,), jnp.float32)
        s, c, dmin = lax.fori_loop(
            0, _CHUNKS, chunk_step,
            (zeros, zeros, jnp.full((16,), _BIG, jnp.float32)), unroll=4)
        c_total = lax.reduce_sum(c, axes=(0,))
        dmin_row = lax.reduce_min(dmin, axes=(0,))
        mrow = jnp.maximum(jnp.float32(_H) - dmin_row, 0.0)

        @pl.when(c_total > jnp.float32(_K) + 0.5)
        def _fallback():
            rowsum = lax.reduce_sum(s, axes=(0,))
            exs = []
            fb = jnp.float32(0.0)
            for r8 in range(_K):
                def estep(jb, carry_e, exs=tuple(exs)):
                    lmin, lj = carry_e
                    jo = jb * 16
                    jvec = jo + lane
                    dd = dist_chunk(q0, q1, q2, sqq, jo)
                    valid = jnp.full((16,), True)
                    for e in exs:
                        valid = jnp.logical_and(valid, jvec != e)
                    ddv = jnp.where(valid, dd, _BIG)
                    better = ddv < lmin
                    lmin = jnp.where(better, ddv, lmin)
                    lj = jnp.where(better, jvec, lj)
                    return lmin, lj

                lmin, lj = lax.fori_loop(
                    0, _CHUNKS, estep,
                    (jnp.full((16,), _BIG, jnp.float32),
                     jnp.full((16,), -1, jnp.int32)))
                mval = lax.reduce_min(lmin, axes=(0,))
                jm = lax.reduce_min(
                    jnp.where(lmin == mval, lj, jnp.int32(2**30)), axes=(0,))
                exs.append(jm)
                if r8 > 0:
                    fb = fb + jnp.maximum(jnp.float32(_H) - mval, 0.0)
            corr_s[0] = corr_s[0] + (fb - (rowsum - mrow))

        return acc + s, msum + mrow

    def group_step(g, carry):
        qbase = seg * _ROWS_PER_W + g * 16
        qv0 = x0v[pl.ds(qbase, 16)]
        qv1 = x1v[pl.ds(qbase, 16)]
        qv2 = x2v[pl.ds(qbase, 16)]
        sqg = sqv[pl.ds(qbase, 16)]
        return lax.fori_loop(
            0, 16, functools.partial(row_step, qv0, qv1, qv2, sqg), carry)

    acc, msum = lax.fori_loop(
        0, _GROUPS, group_step,
        (jnp.zeros((16,), jnp.float32), jnp.float32(0.0)))
    total = lax.reduce_sum(acc, axes=(0,)) - msum + corr_s[0]
    ans_v[...] = jnp.where(lane == 0, total, 0.0)
    pltpu.sync_copy(ans_v, out_hbm.at[wid])


@jax.jit
def kernel(x):
    sq = jnp.sum(x * x, axis=-1).reshape(-1)
    xr = lax.reduce_precision(x, exponent_bits=8, mantissa_bits=7)
    xt = jnp.transpose(xr, (0, 2, 1)).reshape(-1)
    mesh = plsc.VectorSubcoreMesh(
        core_axis_name="c", subcore_axis_name="s",
        num_cores=_NC, num_subcores=_NS)
    part = pl.kernel(
        _body,
        out_type=jax.ShapeDtypeStruct((_NW, 16), jnp.float32),
        mesh=mesh,
        scratch_types=[
            pltpu.VMEM((_N,), jnp.float32),
            pltpu.VMEM((_N,), jnp.float32),
            pltpu.VMEM((_N,), jnp.float32),
            pltpu.VMEM((_N,), jnp.float32),
            pltpu.VMEM((16,), jnp.float32),
            pltpu.SMEM((1,), jnp.float32),
        ],
        compiler_params=pltpu.CompilerParams(needs_layout_passes=False),
    )(xt, sq)
    return jnp.sum(part) / float(_B * _N * (_K - 1))


# speedup vs baseline: 40.1350x; 1.2243x over previous
import functools

import jax
import jax.numpy as jnp
from jax import lax
from jax.experimental import pallas as pl
from jax.experimental.pallas import tpu as pltpu
from jax.experimental.pallas import tpu_sc as plsc

_B = 4
_N = 4096
_H = 0.0005
_K = 8
_NC = 2
_NS = 16
_NW = _NC * _NS
_ROWS_PER_W = (_B * _N) // _NW
_GROUPS = _ROWS_PER_W // 16
_CHUNKS = _N // 16
_TILES_PER_B = _NW // _B
_BIG = 3.0e38


def _body(xr_hbm, sq_hbm, out_hbm, x0v, x1v, x2v, sqv, ans_v, corr_s):
    cid = lax.axis_index("c")
    sid = lax.axis_index("s")
    wid = sid * _NC + cid
    b = wid // _TILES_PER_B
    seg = wid % _TILES_PER_B

    base = b * (3 * _N)
    pltpu.sync_copy(xr_hbm.at[pl.ds(base, _N)], x0v)
    pltpu.sync_copy(xr_hbm.at[pl.ds(base + _N, _N)], x1v)
    pltpu.sync_copy(xr_hbm.at[pl.ds(base + 2 * _N, _N)], x2v)
    pltpu.sync_copy(sq_hbm.at[pl.ds(b * _N, _N)], sqv)

    corr_s[0] = jnp.float32(0.0)
    lane = jax.lax.iota(jnp.int32, 16)

    def dist_chunk(q0d, q1d, q2d, sqq, jo):
        r0 = x0v[pl.ds(jo, 16)]
        r1 = x1v[pl.ds(jo, 16)]
        r2 = x2v[pl.ds(jo, 16)]
        sqj = sqv[pl.ds(jo, 16)]
        dot2 = (q0d * r0 + q1d * r1) + q2d * r2
        dd = (sqq + sqj) - dot2
        return jnp.maximum(dd, 0.0)

    def row_step(qv0, qv1, qv2, sqg, r, carry):
        acc, csum, msum = carry
        ridx = jnp.full((16,), 0, jnp.int32) + r
        q0d = jnp.take_along_axis(qv0, ridx, axis=0) * 2.0
        q1d = jnp.take_along_axis(qv1, ridx, axis=0) * 2.0
        q2d = jnp.take_along_axis(qv2, ridx, axis=0) * 2.0
        sqq = jnp.take_along_axis(sqg, ridx, axis=0)

        def chunk_step(jb, carry_c):
            a, dmin = carry_c
            dd = dist_chunk(q0d, q1d, q2d, sqq, jb * 16)
            a = a + jnp.where(dd < _H, jnp.float32(_H + 0.03125) - dd, 0.0)
            dmin = jnp.minimum(dmin, dd)
            return a, dmin

        a, dmin = lax.fori_loop(
            0, _CHUNKS, chunk_step,
            (jnp.zeros((16,), jnp.float32),
             jnp.full((16,), _BIG, jnp.float32)), unroll=8)
        a_row = lax.reduce_sum(a, axes=(0,))
        c_int = (a_row * 32.0 + 0.25).astype(jnp.int32).astype(jnp.float32)
        c_est = c_int * jnp.float32(0.03125)
        dmin_row = lax.reduce_min(dmin, axes=(0,))
        mrow = jnp.maximum(jnp.float32(_H) - dmin_row, 0.0)

        @pl.when(c_int > jnp.float32(_K) + 0.5)
        def _fallback():
            exs = []
            fb = jnp.float32(0.0)
            for r8 in range(_K):
                def estep(jb, carry_e, exs=tuple(exs)):
                    lmin, lj = carry_e
                    jo = jb * 16
                    jvec = jo + lane
                    dd = dist_chunk(q0d, q1d, q2d, sqq, jo)
                    valid = jnp.full((16,), True)
                    for e in exs:
                        valid = jnp.logical_and(valid, jvec != e)
                    ddv = jnp.where(valid, dd, _BIG)
                    better = ddv < lmin
                    lmin = jnp.where(better, ddv, lmin)
                    lj = jnp.where(better, jvec, lj)
                    return lmin, lj

                lmin, lj = lax.fori_loop(
                    0, _CHUNKS, estep,
                    (jnp.full((16,), _BIG, jnp.float32),
                     jnp.full((16,), -1, jnp.int32)))
                mval = lax.reduce_min(lmin, axes=(0,))
                jm = lax.reduce_min(
                    jnp.where(lmin == mval, lj, jnp.int32(2**30)), axes=(0,))
                exs.append(jm)
                if r8 > 0:
                    fb = fb + jnp.maximum(jnp.float32(_H) - mval, 0.0)
            corr_s[0] = corr_s[0] + (fb - ((a_row - c_est) - mrow))

        return acc + a, csum + c_est, msum + mrow

    def group_step(g, carry):
        qbase = seg * _ROWS_PER_W + g * 16
        qv0 = x0v[pl.ds(qbase, 16)]
        qv1 = x1v[pl.ds(qbase, 16)]
        qv2 = x2v[pl.ds(qbase, 16)]
        sqg = sqv[pl.ds(qbase, 16)]
        return lax.fori_loop(
            0, 16, functools.partial(row_step, qv0, qv1, qv2, sqg), carry)

    acc, csum, msum = lax.fori_loop(
        0, _GROUPS, group_step,
        (jnp.zeros((16,), jnp.float32), jnp.float32(0.0), jnp.float32(0.0)))
    total = lax.reduce_sum(acc, axes=(0,)) - csum - msum + corr_s[0]
    ans_v[...] = jnp.where(lane == 0, total, 0.0)
    pltpu.sync_copy(ans_v, out_hbm.at[wid])


@jax.jit
def kernel(x):
    sq = jnp.sum(x * x, axis=-1).reshape(-1)
    xr = lax.reduce_precision(x, exponent_bits=8, mantissa_bits=7)
    xt = jnp.transpose(xr, (0, 2, 1)).reshape(-1)
    mesh = plsc.VectorSubcoreMesh(
        core_axis_name="c", subcore_axis_name="s",
        num_cores=_NC, num_subcores=_NS)
    part = pl.kernel(
        _body,
        out_type=jax.ShapeDtypeStruct((_NW, 16), jnp.float32),
        mesh=mesh,
        scratch_types=[
            pltpu.VMEM((_N,), jnp.float32),
            pltpu.VMEM((_N,), jnp.float32),
            pltpu.VMEM((_N,), jnp.float32),
            pltpu.VMEM((_N,), jnp.float32),
            pltpu.VMEM((16,), jnp.float32),
            pltpu.SMEM((1,), jnp.float32),
        ],
        compiler_params=pltpu.CompilerParams(needs_layout_passes=False),
    )(xt, sq)
    return jnp.sum(part) / float(_B * _N * (_K - 1))
